# Initial kernel scaffold; baseline (speedup 1.0000x reference)
import jax
import jax.numpy as jnp
from jax.experimental import pallas as pl


def kernel(t_float, X_t_one_hot, edge_index, Wt1, bt1, Wt2, bt2, Wx1, bx1, Wx2, bx2, Wg, bg, g_ln, b_ln, Wo1, bo1, Wo2, bo2):
    raise NotImplementedError("write your pallas kernel here")


# speedup vs baseline: 7.2567x; 7.2567x over previous
import functools

import jax
import jax.numpy as jnp
from jax import lax
from jax.experimental import pallas as pl
from jax.experimental.pallas import tpu as pltpu
from jax.experimental.pallas import tpu_sc as plsc

_N = 10000
_E = 320000
_HX = 128
_HT = 32
_L = 3

_NW = 32
_CHUNK = 128
_CPT = 80
_EPT = _CPT * _CHUNK
_EPAD = _NW * _EPT
_NPAD = 10112
_ZROWS = _NPAD // 16

def _segsum_body(hx, srcp, dstp, zeros_hbm, out, src_v, dst_v, rows0,
                 accum, sem0):
    cid = lax.axis_index("c")
    sid = lax.axis_index("s")
    wid = cid * 16 + sid

    pltpu.sync_copy(srcp.at[wid], src_v)
    pltpu.sync_copy(dstp.at[wid], dst_v)

    pltpu.sync_copy(zeros_hbm, accum.at[pl.ds(sid * _ZROWS, _ZROWS)])
    plsc.subcore_barrier()

    def body(c, _):
        pltpu.async_copy(hx.at[src_v.at[c]], rows0, sem0).wait()
        pltpu.sync_copy(rows0, accum.at[dst_v.at[c]], add=True)
        return 0

    lax.fori_loop(0, _CPT, body, 0)
    plsc.subcore_barrier()

    pltpu.sync_copy(accum.at[pl.ds(sid * _ZROWS, _ZROWS)],
                    out.at[cid, pl.ds(sid * _ZROWS, _ZROWS)])


@functools.cache
def _build_segsum():
    mesh = plsc.VectorSubcoreMesh(core_axis_name="c", subcore_axis_name="s",
                                  num_cores=2, num_subcores=16)
    return pl.kernel(
        _segsum_body,
        out_type=jax.ShapeDtypeStruct((2, _NPAD, _HX), jnp.float32),
        mesh=mesh,
        scratch_types=[
            pltpu.VMEM((_CPT, _CHUNK), jnp.int32),
            pltpu.VMEM((_CPT, _CHUNK), jnp.int32),
            pltpu.VMEM((_CHUNK, _HX), jnp.float32),
            pltpu.VMEM_SHARED((_NPAD, _HX), jnp.float32),
            pltpu.SemaphoreType.DMA,
        ],
    )


_BN = 1000
_GRID = _N // _BN


def _full(shape):
    return pl.BlockSpec(shape, lambda j: tuple(0 for _ in shape))


def _rows(shape):
    return pl.BlockSpec(shape, lambda j: (j,) + tuple(0 for _ in shape[1:]))


def _node_mlp_body(t_ref, wt1_ref, bt1_ref, wt2_ref, bt2_ref,
                   x_ref, wx1_ref, bx1_ref, wx2_ref, bx2_ref,
                   hx_ref, ht_ref):
    t = t_ref[0, 0]
    h1 = jnp.maximum(t * wt1_ref[...] + bt1_ref[...], 0.0)
    ht = jnp.maximum(
        jnp.dot(h1, wt2_ref[...], preferred_element_type=jnp.float32)
        + bt2_ref[...], 0.0)
    ht_ref[...] = ht
    x = x_ref[...]
    a = jnp.maximum(
        jnp.dot(x, wx1_ref[...], preferred_element_type=jnp.float32)
        + bx1_ref[...], 0.0)
    hx_ref[...] = jnp.maximum(
        jnp.dot(a, wx2_ref[...], preferred_element_type=jnp.float32)
        + bx2_ref[...], 0.0)


def _node_mlp(t2, wt1, bt1, wt2, bt2, x, wx1, bx1, wx2, bx2):
    return pl.pallas_call(
        _node_mlp_body,
        grid=(_GRID,),
        in_specs=[
            _full((1, 1)), _full((1, _HT)), _full((1, _HT)),
            _full((_HT, _HT)), _full((1, _HT)),
            _rows((_BN, _HX)), _full((_HX, _HX)), _full((1, _HX)),
            _full((_HX, _HX)), _full((1, _HX)),
        ],
        out_specs=[_rows((_BN, _HX)), _full((1, _HT))],
        out_shape=[
            jax.ShapeDtypeStruct((_N, _HX), jnp.float32),
            jax.ShapeDtypeStruct((1, _HT), jnp.float32),
        ],
    )(t2, wt1, bt1, wt2, bt2, x, wx1, bx1, wx2, bx2)


def _layer_body(p_ref, ht_ref, wgx_ref, wgt_ref, bg_ref, g_ref, b_ref,
                out_ref):
    aggr = p_ref[0] + p_ref[1]
    tvec = jnp.dot(ht_ref[...], wgt_ref[...],
                   preferred_element_type=jnp.float32)
    h = jnp.maximum(
        jnp.dot(aggr, wgx_ref[...], preferred_element_type=jnp.float32)
        + tvec + bg_ref[...], 0.0)
    m = jnp.mean(h, axis=-1, keepdims=True)
    c = h - m
    v = jnp.mean(c * c, axis=-1, keepdims=True)
    out_ref[...] = c * jax.lax.rsqrt(v + 1e-5) * g_ref[...] + b_ref[...]


def _layer_update(p, ht, wgx, wgt, bg_i, g_i, b_i):
    return pl.pallas_call(
        _layer_body,
        grid=(_GRID,),
        in_specs=[
            pl.BlockSpec((2, _BN, _HX), lambda j: (0, j, 0)),
            _full((1, _HT)), _full((_HX, _HX)), _full((_HT, _HX)),
            _full((1, _HX)), _full((1, _HX)), _full((1, _HX)),
        ],
        out_specs=_rows((_BN, _HX)),
        out_shape=jax.ShapeDtypeStruct((_N, _HX), jnp.float32),
    )(p, ht, wgx, wgt, bg_i, g_i, b_i)


def _readout_body(h0_ref, h1_ref, h2_ref, h3_ref, ht_ref,
                  w1x_ref, w1t_ref, bo1_ref, wo2_ref, bo2_ref, out_ref):
    hcat = jnp.concatenate(
        [h0_ref[...], h1_ref[...], h2_ref[...], h3_ref[...]], axis=1)
    tvec = jnp.dot(ht_ref[...], w1t_ref[...],
                   preferred_element_type=jnp.float32)
    y = jnp.maximum(
        jnp.dot(hcat, w1x_ref[...], preferred_element_type=jnp.float32)
        + tvec + bo1_ref[...], 0.0)
    out_ref[...] = (
        jnp.dot(y, wo2_ref[...], preferred_element_type=jnp.float32)
        + bo2_ref[...])


def _readout(h0, h1, h2, h3, ht, w1x, w1t, bo1, wo2, bo2):
    cat = 4 * _HX + _HT
    return pl.pallas_call(
        _readout_body,
        grid=(_GRID,),
        in_specs=[
            _rows((_BN, _HX)), _rows((_BN, _HX)), _rows((_BN, _HX)),
            _rows((_BN, _HX)), _full((1, _HT)),
            _full((4 * _HX, cat)), _full((_HT, cat)), _full((1, cat)),
            _full((cat, _HX)), _full((1, _HX)),
        ],
        out_specs=_rows((_BN, _HX)),
        out_shape=jax.ShapeDtypeStruct((_N, _HX), jnp.float32),
    )(h0, h1, h2, h3, ht, w1x, w1t, bo1, wo2, bo2)


def kernel(t_float, X_t_one_hot, edge_index, Wt1, bt1, Wt2, bt2, Wx1, bx1,
           Wx2, bx2, Wg, bg, g_ln, b_ln, Wo1, bo1, Wo2, bo2):
    src = edge_index[0].astype(jnp.int32)
    dst = edge_index[1].astype(jnp.int32)
    pad = _EPAD - _E
    it = jnp.arange(pad, dtype=jnp.int32)
    srcp = jnp.concatenate([src, it % _N])
    dstp = jnp.concatenate([dst, _N + it % (_NPAD - _N)])
    srcp = srcp.reshape(_NW, _CPT, _CHUNK)
    dstp = dstp.reshape(_NW, _CPT, _CHUNK)
    zeros_hbm = jnp.zeros((_ZROWS, _HX), jnp.float32)

    t2 = t_float.reshape(1, 1)
    hX, ht = _node_mlp(t2, Wt1, bt1.reshape(1, _HT), Wt2, bt2.reshape(1, _HT),
                       X_t_one_hot, Wx1, bx1.reshape(1, _HX), Wx2,
                       bx2.reshape(1, _HX))

    hs = [hX]
    for i in range(_L):
        p = _build_segsum()(hX, srcp, dstp, zeros_hbm)
        hX = _layer_update(p, ht, Wg[i, :_HX], Wg[i, _HX:],
                           bg[i].reshape(1, _HX), g_ln[i].reshape(1, _HX),
                           b_ln[i].reshape(1, _HX))
        hs.append(hX)

    cat = 4 * _HX + _HT
    out = _readout(hs[0], hs[1], hs[2], hs[3], ht,
                   Wo1[:4 * _HX], Wo1[4 * _HX:], bo1.reshape(1, cat),
                   Wo2, bo2.reshape(1, _HX))
    return out
